# baseline (speedup 1.0000x reference)
import jax
import jax.numpy as jnp
from jax.experimental import pallas as pl

_NUM_WINDOWS = 4
_CHUNK = 256
_LANES = 128


def _kmeans_body(x_ref, c_ref, cn_ref, out_ref):
    xb = x_ref[...]
    bn = xb.shape[0]
    k = c_ref.shape[1]
    kw = k // _NUM_WINDOWS
    xsq = jnp.sum(xb * xb, axis=1, keepdims=True)
    acc_v = jnp.full((bn,), jnp.inf, dtype=jnp.float32)
    acc_i = jnp.zeros((bn,), dtype=jnp.int32)
    lane_iota = jax.lax.broadcasted_iota(jnp.int32, (bn, _LANES), 1)
    for w in range(_NUM_WINDOWS):
        run_v = jnp.full((bn, _LANES), jnp.inf, dtype=jnp.float32)
        run_t = jnp.zeros((bn, _LANES), dtype=jnp.int32)
        for j in range(kw // _CHUNK):
            base = w * kw + j * _CHUNK
            mm = jnp.dot(xb, c_ref[:, base:base + _CHUNK],
                         preferred_element_type=jnp.float32)
            for sub in range(_CHUNK // _LANES):
                lo = sub * _LANES
                dj = xsq + mm[:, lo:lo + _LANES] + cn_ref[:, base + lo:base + lo + _LANES]
                pred = dj < run_v
                run_v = jnp.where(pred, dj, run_v)
                run_t = jnp.where(pred, j * (_CHUNK // _LANES) + sub, run_t)
        wmin = jnp.min(run_v, axis=1)
        packed = run_t * _LANES + lane_iota
        widx = jnp.min(jnp.where(run_v == wmin[:, None], packed, kw),
                       axis=1) + w * kw
        take = wmin < acc_v
        acc_i = jnp.where(take, widx, acc_i)
        acc_v = jnp.where(take, wmin, acc_v).astype(jnp.bfloat16).astype(jnp.float32)
    out_ref[...] = acc_i


def kernel(x, centroids, centroid_norm):
    n, d = x.shape
    k = centroids.shape[1]
    bn = 4096
    cneg = centroids * (-2.0)
    return pl.pallas_call(
        _kmeans_body,
        grid=(n // bn,),
        in_specs=[
            pl.BlockSpec((bn, d), lambda i: (i, 0)),
            pl.BlockSpec((d, k), lambda i: (0, 0)),
            pl.BlockSpec((1, k), lambda i: (0, 0)),
        ],
        out_specs=pl.BlockSpec((bn,), lambda i: (i,)),
        out_shape=jax.ShapeDtypeStruct((n,), jnp.int32),
    )(x, cneg, centroid_norm)


# speedup vs baseline: 1.0024x; 1.0024x over previous
import jax
import jax.numpy as jnp
from jax.experimental import pallas as pl

_NUM_WINDOWS = 4
_CHUNK = 512
_LANES = 128


def _kmeans_body(x_ref, c_ref, cn_ref, out_ref):
    xb = x_ref[...]
    bn = xb.shape[0]
    k = c_ref.shape[1]
    kw = k // _NUM_WINDOWS
    xsq = jnp.sum(xb * xb, axis=1, keepdims=True)
    acc_v = jnp.full((bn,), jnp.inf, dtype=jnp.float32)
    acc_i = jnp.zeros((bn,), dtype=jnp.int32)
    lane_iota = jax.lax.broadcasted_iota(jnp.int32, (bn, _LANES), 1)
    for w in range(_NUM_WINDOWS):
        run_v = jnp.full((bn, _LANES), jnp.inf, dtype=jnp.float32)
        run_t = jnp.zeros((bn, _LANES), dtype=jnp.int32)
        for j in range(kw // _CHUNK):
            base = w * kw + j * _CHUNK
            mm = jnp.dot(xb, c_ref[:, base:base + _CHUNK],
                         preferred_element_type=jnp.float32)
            for sub in range(_CHUNK // _LANES):
                lo = sub * _LANES
                dj = xsq + mm[:, lo:lo + _LANES] + cn_ref[:, base + lo:base + lo + _LANES]
                pred = dj < run_v
                run_v = jnp.where(pred, dj, run_v)
                run_t = jnp.where(pred, j * (_CHUNK // _LANES) + sub, run_t)
        wmin = jnp.min(run_v, axis=1)
        packed = run_t * _LANES + lane_iota
        widx = jnp.min(jnp.where(run_v == wmin[:, None], packed, kw),
                       axis=1) + w * kw
        take = wmin < acc_v
        acc_i = jnp.where(take, widx, acc_i)
        acc_v = jnp.where(take, wmin, acc_v).astype(jnp.bfloat16).astype(jnp.float32)
    out_ref[...] = acc_i


def kernel(x, centroids, centroid_norm):
    n, d = x.shape
    k = centroids.shape[1]
    bn = min(2048, n)
    cneg = centroids * (-2.0)
    return pl.pallas_call(
        _kmeans_body,
        grid=(n // bn,),
        in_specs=[
            pl.BlockSpec((bn, d), lambda i: (i, 0)),
            pl.BlockSpec((d, k), lambda i: (0, 0)),
            pl.BlockSpec((1, k), lambda i: (0, 0)),
        ],
        out_specs=pl.BlockSpec((bn,), lambda i: (i,)),
        out_shape=jax.ShapeDtypeStruct((n,), jnp.int32),
    )(x, cneg, centroid_norm)
